# Initial kernel scaffold; baseline (speedup 1.0000x reference)
import jax
import jax.numpy as jnp
from jax.experimental import pallas as pl


def kernel(x, edge_index, W, b, eps):
    raise NotImplementedError("write your pallas kernel here")


# speedup vs baseline: 5.8042x; 5.8042x over previous
import functools

import jax
import jax.numpy as jnp
from jax import lax
from jax.experimental import pallas as pl
from jax.experimental.pallas import tpu as pltpu
from jax.experimental.pallas import tpu_sc as plsc

N = 10000
C = 128
K = 32
G = 4
NPAD = 10240
NW = 32
NPW = NPAD // NW
CHUNK_NODES = 4
CHUNKS = NPW // CHUNK_NODES
NLANE = 16
NV = C // NLANE


def _transpose_body(x_ref, o_ref):
    o_ref[...] = x_ref[...].T


def _transpose_cn_to_nc(x_pad):
    return pl.pallas_call(
        _transpose_body,
        grid=(NPAD // 128,),
        in_specs=[pl.BlockSpec((C, 128), lambda i: (0, i))],
        out_specs=pl.BlockSpec((128, C), lambda i: (i, 0)),
        out_shape=jax.ShapeDtypeStruct((NPAD, C), jnp.float32),
    )(x_pad)


def _sc_gather_sum(xT_hbm, idx_hbm, out_hbm, idx_v, buf0, buf1, out_v,
                   sem0, sem1):
    wid = lax.axis_index("s") * 2 + lax.axis_index("c")
    pltpu.sync_copy(idx_hbm.at[wid], idx_v)

    def gather(c, buf, sem):
        pltpu.make_async_copy(xT_hbm.at[idx_v.at[c]], buf, sem).start()

    def wait(buf, sem):
        pltpu.make_async_copy(xT_hbm.at[idx_v.at[0]], buf, sem).wait()

    def compute_chunk(buf, c):
        for i in range(CHUNK_NODES):
            def rbody(q, accs, _i=i):
                base = _i * K + q * 4
                for rr in range(4):
                    row = base + rr
                    accs = tuple(
                        accs[j] + buf[row, pl.ds(j * NLANE, NLANE)]
                        for j in range(NV)
                    )
                return accs
            accs0 = tuple(jnp.zeros((NLANE,), jnp.float32) for _ in range(NV))
            accs = lax.fori_loop(0, K // 4, rbody, accs0)
            node = c * CHUNK_NODES + i
            for j in range(NV):
                out_v[node, pl.ds(j * NLANE, NLANE)] = accs[j]

    gather(0, buf0, sem0)

    def pair(p, carry):
        gather(2 * p + 1, buf1, sem1)
        wait(buf0, sem0)
        compute_chunk(buf0, 2 * p)

        @pl.when(p < CHUNKS // 2 - 1)
        def _():
            gather(2 * p + 2, buf0, sem0)

        wait(buf1, sem1)
        compute_chunk(buf1, 2 * p + 1)
        return carry

    lax.fori_loop(0, CHUNKS // 2, pair, 0)

    pltpu.sync_copy(out_v, out_hbm.at[pl.ds(wid * NPW, NPW)])


def _neighbor_sum(xT, idx3):
    mesh = plsc.VectorSubcoreMesh(core_axis_name="c", subcore_axis_name="s",
                                  num_cores=2, num_subcores=16)
    kern = functools.partial(
        pl.kernel,
        out_type=jax.ShapeDtypeStruct((NPAD, C), jnp.float32),
        mesh=mesh,
        scratch_types=[
            pltpu.VMEM((CHUNKS, 128), jnp.int32),
            pltpu.VMEM((CHUNK_NODES * K, C), jnp.float32),
            pltpu.VMEM((CHUNK_NODES * K, C), jnp.float32),
            pltpu.VMEM((NPW, C), jnp.float32),
            pltpu.SemaphoreType.DMA,
            pltpu.SemaphoreType.DMA,
        ],
    )(_sc_gather_sum)
    return kern(xT, idx3)


def _conv_body(eps_ref, xT_ref, xj_ref, W_ref, b_ref, o_ref):
    scale = 1.0 + eps_ref[0]
    h = scale * xT_ref[...] + xj_ref[...]
    y = lax.dot_general(W_ref[...], h, (((1,), (1,)), ((), ())),
                        preferred_element_type=jnp.float32)
    o_ref[...] = jnp.maximum(y + b_ref[...], 0.0)


def _gin_update(eps, xT, xj, W_bd, b):
    return pl.pallas_call(
        _conv_body,
        grid=(NPAD // 128,),
        in_specs=[
            pl.BlockSpec(memory_space=pltpu.SMEM),
            pl.BlockSpec((128, C), lambda i: (i, 0)),
            pl.BlockSpec((128, C), lambda i: (i, 0)),
            pl.BlockSpec((C, C), lambda i: (0, 0)),
            pl.BlockSpec((C, 1), lambda i: (0, 0)),
        ],
        out_specs=pl.BlockSpec((C, 128), lambda i: (0, i)),
        out_shape=jax.ShapeDtypeStruct((C, NPAD), jnp.float32),
    )(eps, xT, xj, W_bd, b)


def kernel(x, edge_index, W, b, eps):
    x_sq = x[0, :, :, 0]
    x_pad = jnp.pad(x_sq, ((0, 0), (0, NPAD - N)))
    idx = edge_index[0, 0]
    idx_pad = jnp.pad(idx, ((0, NPAD - N), (0, 0)))
    idx3 = idx_pad.reshape(NW, CHUNKS, 128)

    Wg = W[:, :, 0, 0]
    W_bd = jnp.zeros((C, C), jnp.float32)
    gs = C // G
    for g in range(G):
        W_bd = W_bd.at[g * gs:(g + 1) * gs, g * gs:(g + 1) * gs].set(
            Wg[g * gs:(g + 1) * gs, :])

    xT = _transpose_cn_to_nc(x_pad)
    xj = _neighbor_sum(xT, idx3)
    out = _gin_update(eps, xT, xj, W_bd, b[:, None])
    return out[:, :N][None, :, :, None]


# speedup vs baseline: 5.9173x; 1.0195x over previous
import functools

import jax
import jax.numpy as jnp
from jax import lax
from jax.experimental import pallas as pl
from jax.experimental.pallas import tpu as pltpu
from jax.experimental.pallas import tpu_sc as plsc

N = 10000
C = 128
K = 32
G = 4
NPAD = 10240
NW = 32
NPW = NPAD // NW
CHUNK_NODES = 4
CHUNKS = NPW // CHUNK_NODES
NLANE = 16
NV = C // NLANE


def _transpose_body(x_ref, o_ref):
    o_ref[...] = x_ref[...].T


def _transpose_cn_to_nc(x_pad):
    return pl.pallas_call(
        _transpose_body,
        grid=(NPAD // 128,),
        in_specs=[pl.BlockSpec((C, 128), lambda i: (0, i))],
        out_specs=pl.BlockSpec((128, C), lambda i: (i, 0)),
        out_shape=jax.ShapeDtypeStruct((NPAD, C), jnp.float32),
    )(x_pad)


NBUF = 4


def _sc_gather_sum(xT_hbm, idx_hbm, out_hbm, idx_v, buf0, buf1, buf2, buf3,
                   out_v, sem0, sem1, sem2, sem3):
    bufs = (buf0, buf1, buf2, buf3)
    sems = (sem0, sem1, sem2, sem3)
    wid = lax.axis_index("s") * 2 + lax.axis_index("c")
    pltpu.sync_copy(idx_hbm.at[wid], idx_v)

    def gather(c, buf, sem):
        pltpu.make_async_copy(xT_hbm.at[idx_v.at[c]], buf, sem).start()

    def wait(buf, sem):
        pltpu.make_async_copy(xT_hbm.at[idx_v.at[0]], buf, sem).wait()

    def compute_chunk(buf, c):
        for i in range(CHUNK_NODES):
            def rbody(q, accs, _i=i):
                base = _i * K + q * 4
                for rr in range(4):
                    row = base + rr
                    accs = tuple(
                        accs[j] + buf[row, pl.ds(j * NLANE, NLANE)]
                        for j in range(NV)
                    )
                return accs
            accs0 = tuple(jnp.zeros((NLANE,), jnp.float32) for _ in range(NV))
            accs = lax.fori_loop(0, K // 4, rbody, accs0)
            node = c * CHUNK_NODES + i
            for j in range(NV):
                out_v[node, pl.ds(j * NLANE, NLANE)] = accs[j]

    for b in range(NBUF):
        gather(b, bufs[b], sems[b])

    def step(p, carry):
        for b in range(NBUF):
            c = NBUF * p + b
            wait(bufs[b], sems[b])
            compute_chunk(bufs[b], c)

            @pl.when(p < CHUNKS // NBUF - 1)
            def _(b=b, c=c):
                gather(c + NBUF, bufs[b], sems[b])
        return carry

    lax.fori_loop(0, CHUNKS // NBUF, step, 0)

    pltpu.sync_copy(out_v, out_hbm.at[pl.ds(wid * NPW, NPW)])


def _neighbor_sum(xT, idx3):
    mesh = plsc.VectorSubcoreMesh(core_axis_name="c", subcore_axis_name="s",
                                  num_cores=2, num_subcores=16)
    kern = functools.partial(
        pl.kernel,
        out_type=jax.ShapeDtypeStruct((NPAD, C), jnp.float32),
        mesh=mesh,
        scratch_types=(
            [pltpu.VMEM((CHUNKS, 128), jnp.int32)]
            + [pltpu.VMEM((CHUNK_NODES * K, C), jnp.float32)
               for _ in range(NBUF)]
            + [pltpu.VMEM((NPW, C), jnp.float32)]
            + [pltpu.SemaphoreType.DMA for _ in range(NBUF)]
        ),
    )(_sc_gather_sum)
    return kern(xT, idx3)


def _conv_body(eps_ref, xT_ref, xj_ref, W_ref, b_ref, o_ref):
    scale = 1.0 + eps_ref[0]
    h = scale * xT_ref[...] + xj_ref[...]
    y = lax.dot_general(W_ref[...], h, (((1,), (1,)), ((), ())),
                        preferred_element_type=jnp.float32)
    o_ref[...] = jnp.maximum(y + b_ref[...], 0.0)


def _gin_update(eps, xT, xj, W_bd, b):
    return pl.pallas_call(
        _conv_body,
        grid=(NPAD // 128,),
        in_specs=[
            pl.BlockSpec(memory_space=pltpu.SMEM),
            pl.BlockSpec((128, C), lambda i: (i, 0)),
            pl.BlockSpec((128, C), lambda i: (i, 0)),
            pl.BlockSpec((C, C), lambda i: (0, 0)),
            pl.BlockSpec((C, 1), lambda i: (0, 0)),
        ],
        out_specs=pl.BlockSpec((C, 128), lambda i: (0, i)),
        out_shape=jax.ShapeDtypeStruct((C, NPAD), jnp.float32),
    )(eps, xT, xj, W_bd, b)


def kernel(x, edge_index, W, b, eps):
    x_sq = x[0, :, :, 0]
    x_pad = jnp.pad(x_sq, ((0, 0), (0, NPAD - N)))
    idx = edge_index[0, 0]
    idx_pad = jnp.pad(idx, ((0, NPAD - N), (0, 0)))
    idx3 = idx_pad.reshape(NW, CHUNKS, 128)

    Wg = W[:, :, 0, 0]
    W_bd = jnp.zeros((C, C), jnp.float32)
    gs = C // G
    for g in range(G):
        W_bd = W_bd.at[g * gs:(g + 1) * gs, g * gs:(g + 1) * gs].set(
            Wg[g * gs:(g + 1) * gs, :])

    xT = _transpose_cn_to_nc(x_pad)
    xj = _neighbor_sum(xT, idx3)
    out = _gin_update(eps, xT, xj, W_bd, b[:, None])
    return out[:, :N][None, :, :, None]
